# baseline (speedup 1.0000x reference)
import functools

import jax
import jax.numpy as jnp
from jax import lax
from jax.experimental import pallas as pl
from jax.experimental.pallas import tpu as pltpu
from jax.experimental.pallas import tpu_sc as plsc

B, N, C, K = 4, 1024, 256, 16
CP = 16
QW = 128
CW = 128
RB = 256
NB = 256
NTOT = B * N
NKTOT = B * N * K
GCH = 128

_F32 = jnp.float32
_S2 = 0.7071067811865476


def _g(y):
    return y * (1.0 + lax.erf(y))


def _prep_body(pw2_ref, mw1_ref, mb1_ref, pb2_ref, w_ref, wpc_ref, c0_ref):
    mw1c = mw1_ref[2 * C:3 * C, :]
    w_ref[...] = 0.5 * jnp.dot(pw2_ref[...], mw1c,
                               preferred_element_type=_F32)
    wpc_ref[...] = _S2 * (mw1_ref[0:C, :] - mw1_ref[C:2 * C, :])
    c0_ref[...] = _S2 * (mb1_ref[...] + jnp.dot(pb2_ref[...], mw1c,
                                                preferred_element_type=_F32))


def _ka_body(tok_ref, call_ref, cblk_ref, n1g_ref, n1b_ref, wpc_ref, wq_ref,
             c0_ref, p_ref, t_ref, c_ref, idx_ref):
    b = pl.program_id(0)
    r = pl.program_id(1)

    call = call_ref[0]
    crb = cblk_ref[0]

    x = tok_ref[0]
    mu = jnp.mean(x, axis=1, keepdims=True)
    var = jnp.mean((x - mu) ** 2, axis=1, keepdims=True)
    tn = (x - mu) / jnp.sqrt(var + 1e-5) * n1g_ref[...] + n1b_ref[...]
    p_ref[0] = jnp.dot(tn, wpc_ref[...], preferred_element_type=_F32) \
        + c0_ref[...]
    q = jnp.dot(tn, wq_ref[...], preferred_element_type=_F32)
    hi = lax.bitcast_convert_type(
        q[:, 0:QW].astype(jnp.bfloat16).astype(_F32), jnp.int32)
    lo = lax.bitcast_convert_type(
        q[:, QW:C].astype(jnp.bfloat16).astype(_F32), jnp.int32)
    t_ref[0] = hi | lax.shift_right_logical(lo, 16)
    c_ref[0] = jnp.concatenate(
        [crb, jnp.zeros((RB, CW - CP), _F32)], axis=1)
    sqc = jnp.sum(crb * crb, axis=1, keepdims=True)
    ones_row = jnp.ones((1, CP), _F32)
    sqr = lax.dot_general(ones_row, call * call,
                          (((1,), (1,)), ((), ())),
                          preferred_element_type=_F32)
    cct = lax.dot_general(crb, call, (((1,), (1,)), ((), ())),
                          preferred_element_type=_F32)
    d2 = sqc + sqr - 2.0 * cct

    rows = lax.broadcasted_iota(jnp.int32, (RB, N), 0) + r * RB
    cols = lax.broadcasted_iota(jnp.int32, (RB, N), 1)
    d2 = jnp.where(rows == cols, 1e9, d2)
    bits = lax.bitcast_convert_type(d2, jnp.int32)
    keys = (bits & jnp.int32(-1024)) | cols

    fw = N // 4
    x0, x1 = keys[:, 0:fw], keys[:, fw:2 * fw]
    x2, x3 = keys[:, 2 * fw:3 * fw], keys[:, 3 * fw:4 * fw]
    a0, a1 = jnp.minimum(x0, x1), jnp.maximum(x0, x1)
    a2, a3 = jnp.minimum(x2, x3), jnp.maximum(x2, x3)
    b0, b2 = jnp.minimum(a0, a2), jnp.maximum(a0, a2)
    b1, b3 = jnp.minimum(a1, a3), jnp.maximum(a1, a3)
    c1, c2 = jnp.minimum(b1, b2), jnp.maximum(b1, b2)
    cur, n1, n2, n3 = b0, c1, c2, b3

    imax = jnp.int32(2147483647)
    lane_k = lax.broadcasted_iota(jnp.int32, (RB, K), 1)
    acc = jnp.zeros((RB, K), jnp.int32)
    for t in range(K):
        m = jnp.min(cur, axis=1, keepdims=True)
        acc = jnp.where(lane_k == t, (m & 1023) + b * N, acc)
        eq = cur == m
        cur = jnp.where(eq, n1, cur)
        n1 = jnp.where(eq, n2, n1)
        n2 = jnp.where(eq, n3, n2)
        n3 = jnp.where(eq, imax, n3)
    idx_ref[0] = acc


@functools.lru_cache(maxsize=None)
def _make_sc_gather(nktot):
    info = plsc.get_sparse_core_info()
    nc, ns = info.num_cores, info.num_subcores
    nw = nc * ns
    b_per_w = nktot // nw
    nch = b_per_w // GCH
    mesh = plsc.VectorSubcoreMesh(core_axis_name="c", subcore_axis_name="s")

    @functools.partial(
        pl.kernel, mesh=mesh,
        out_type=[jax.ShapeDtypeStruct((nktot, QW), jnp.int32),
                  jax.ShapeDtypeStruct((nktot, CW), _F32)],
        scratch_types=[
            pltpu.VMEM((GCH,), jnp.int32),
            pltpu.VMEM((GCH,), jnp.int32),
            pltpu.VMEM((GCH, QW), jnp.int32),
            pltpu.VMEM((GCH, QW), jnp.int32),
            pltpu.VMEM((GCH, CW), _F32),
            pltpu.VMEM((GCH, CW), _F32),
            pltpu.SemaphoreType.DMA,
            pltpu.SemaphoreType.DMA,
            pltpu.SemaphoreType.DMA,
            pltpu.SemaphoreType.DMA,
        ],
    )
    def gather_k(qtab, ctab, idx, qg, cg, idx0, idx1, qb0, qb1, cb0, cb1,
                 sg0, sg1, so0, so1):
        wid = lax.axis_index("s") * nc + lax.axis_index("c")
        base = wid * b_per_w
        idxb, qb, cb = [idx0, idx1], [qb0, qb1], [cb0, cb1]
        sg, so = [sg0, sg1], [so0, so1]
        gq = [None, None]
        gc = [None, None]
        oq = [None, None]
        oc = [None, None]

        def start_out(j, off):
            gq[j].wait()
            gc[j].wait()
            oq[j] = pltpu.async_copy(qb[j], qg.at[pl.ds(off, GCH)], so[j])
            oc[j] = pltpu.async_copy(cb[j], cg.at[pl.ds(off, GCH)], so[j])

        for ch in range(nch):
            bi = ch % 2
            if oq[bi] is not None:
                oq[bi].wait()
                oc[bi].wait()
            pltpu.sync_copy(idx.at[pl.ds(base + ch * GCH, GCH)], idxb[bi])
            gq[bi] = pltpu.async_copy(qtab.at[idxb[bi]], qb[bi], sg[bi])
            gc[bi] = pltpu.async_copy(ctab.at[idxb[bi]], cb[bi], sg[bi])
            if ch >= 1:
                start_out((ch - 1) % 2, base + (ch - 1) * GCH)
        last = (nch - 1) % 2
        start_out(last, base + (nch - 1) * GCH)
        oq[0].wait()
        oc[0].wait()
        oq[1].wait()
        oc[1].wait()

    return gather_k


def _sc_gather(qtab, ctab, idxf):
    return _make_sc_gather(idxf.shape[0])(qtab, ctab, idxf)


def _kc_body(tok_ref, p_ref, cpd_ref, qg_ref, cg_ref, pw1p_ref, w_ref,
             mw2_ref, fw1_ref, fw2_ref, pb1_ref, mb2_ref, n2g_ref, n2b_ref,
             fb1_ref, fb2_ref, out_ref):
    cg3 = cg_ref[:, 0:CP].reshape(NB, K, CP)
    dxyz = (cg3 - cpd_ref[...][:, None, :]).reshape(NB * K, CP)
    posh = _g(jnp.dot(dxyz, pw1p_ref[...], preferred_element_type=_F32)
              + pb1_ref[...])
    u = jnp.dot(posh, w_ref[...], preferred_element_type=_F32)
    v = qg_ref[...]
    qhi = lax.bitcast_convert_type(v & jnp.int32(-65536), _F32)
    qlo = lax.bitcast_convert_type(lax.shift_left(v, 16), _F32)
    qg = jnp.concatenate([qhi, qlo], axis=1)
    h3 = _g(u.reshape(NB, K, C) + qg.reshape(NB, K, C)
            + p_ref[...][:, None, :])
    hs = jnp.sum(h3, axis=1)
    t = tok_ref[...] + jnp.dot(hs, mw2_ref[...],
                               preferred_element_type=_F32) + mb2_ref[...]

    mu = jnp.mean(t, axis=1, keepdims=True)
    var = jnp.mean((t - mu) ** 2, axis=1, keepdims=True)
    h = (t - mu) / jnp.sqrt(var + 1e-5) * n2g_ref[...] + n2b_ref[...]
    f = _g(jnp.dot(h, fw1_ref[...], preferred_element_type=_F32)
           + fb1_ref[...])
    out_ref[...] = t + jnp.dot(f, fw2_ref[...],
                               preferred_element_type=_F32) + fb2_ref[...]


def kernel(tokens, centers, n1_g, n1_b, pw1, pb1, pw2, pb2, mw1, mb1, mw2,
           mb2, n2_g, n2_b, fw1, fb1, fw2, fb2):
    row = lambda v: v.reshape(1, -1)
    cpad = jnp.pad(centers, ((0, 0), (0, 0), (0, CP - 3)))
    pw1p = _S2 * jnp.pad(pw1, ((0, CP - 3), (0, 0)))
    pb1_s = _S2 * pb1
    wq = _S2 * mw1[C:2 * C]
    mw2_s = (_S2 / K) * mw2
    fw1_s = _S2 * fw1
    fb1_s = _S2 * fb1
    fw2_s = _S2 * fw2

    w_fused, wpc, c0 = pl.pallas_call(
        _prep_body,
        out_shape=[jax.ShapeDtypeStruct((C, C), _F32),
                   jax.ShapeDtypeStruct((C, C), _F32),
                   jax.ShapeDtypeStruct((1, C), _F32)],
    )(pw2, mw1, row(mb1), row(pb2))

    BH = 1
    nth = BH * N
    nkh = nth * K

    def run_half(tok_h, cpad_h):
        full = lambda s: pl.BlockSpec(s, lambda b, r: (0, 0))
        p_arr, t_arr, c_tab, idx = pl.pallas_call(
            _ka_body,
            grid=(BH, N // RB),
            in_specs=[
                pl.BlockSpec((1, RB, C), lambda b, r: (b, r, 0)),
                pl.BlockSpec((1, N, CP), lambda b, r: (b, 0, 0)),
                pl.BlockSpec((1, RB, CP), lambda b, r: (b, r, 0)),
                full((1, C)), full((1, C)),
                full((C, C)), full((C, C)), full((1, C)),
            ],
            out_specs=[
                pl.BlockSpec((1, RB, C), lambda b, r: (b, r, 0)),
                pl.BlockSpec((1, RB, QW), lambda b, r: (b, r, 0)),
                pl.BlockSpec((1, RB, CW), lambda b, r: (b, r, 0)),
                pl.BlockSpec((1, RB, K), lambda b, r: (b, r, 0)),
            ],
            out_shape=[jax.ShapeDtypeStruct((BH, N, C), _F32),
                       jax.ShapeDtypeStruct((BH, N, QW), jnp.int32),
                       jax.ShapeDtypeStruct((BH, N, CW), _F32),
                       jax.ShapeDtypeStruct((BH, N, K), jnp.int32)],
        )(tok_h, cpad_h, cpad_h, row(n1_g), row(n1_b), wpc, wq, c0)

        qg, cg = _sc_gather(t_arr.reshape(nth, QW),
                            c_tab.reshape(nth, CW), idx.reshape(nkh))

        wfull = lambda s: pl.BlockSpec(s, lambda i: (0, 0))
        out = pl.pallas_call(
            _kc_body,
            grid=(nth // NB,),
            in_specs=[
                pl.BlockSpec((NB, C), lambda i: (i, 0)),
                pl.BlockSpec((NB, C), lambda i: (i, 0)),
                pl.BlockSpec((NB, CP), lambda i: (i, 0)),
                pl.BlockSpec((NB * K, QW), lambda i: (i, 0)),
                pl.BlockSpec((NB * K, CW), lambda i: (i, 0)),
                wfull((CP, C)), wfull((C, C)), wfull((C, C)),
                wfull((C, 4 * C)), wfull((4 * C, C)),
                wfull((1, C)), wfull((1, C)), wfull((1, C)), wfull((1, C)),
                wfull((1, 4 * C)), wfull((1, C)),
            ],
            out_specs=pl.BlockSpec((NB, C), lambda i: (i, 0)),
            out_shape=jax.ShapeDtypeStruct((nth, C), _F32),
        )(tok_h.reshape(nth, C), p_arr.reshape(nth, C),
          cpad_h.reshape(nth, CP), qg, cg, pw1p, w_fused, mw2_s, fw1_s,
          fw2_s, row(pb1_s), row(mb2), row(n2_g), row(n2_b), row(fb1_s),
          row(fb2))
        return out

    parts = [run_half(tokens[i:i + BH], cpad[i:i + BH])
             for i in range(0, B, BH)]
    return jnp.concatenate(parts, axis=0).reshape(B, N, C)


# speedup vs baseline: 1.0236x; 1.0236x over previous
import functools

import jax
import jax.numpy as jnp
from jax import lax
from jax.experimental import pallas as pl
from jax.experimental.pallas import tpu as pltpu
from jax.experimental.pallas import tpu_sc as plsc

B, N, C, K = 4, 1024, 256, 16
CP = 16
QW = 128
CW = 128
RB = 256
NB = 256
NTOT = B * N
NKTOT = B * N * K
GCH = 128

_F32 = jnp.float32
_S2 = 0.7071067811865476


def _g(y):
    return y * (1.0 + lax.erf(y))


def _prep_body(pw2_ref, mw1_ref, mb1_ref, pb2_ref, w_ref, wpc_ref, c0_ref):
    mw1c = mw1_ref[2 * C:3 * C, :]
    w_ref[...] = 0.5 * jnp.dot(pw2_ref[...], mw1c,
                               preferred_element_type=_F32)
    wpc_ref[...] = _S2 * (mw1_ref[0:C, :] - mw1_ref[C:2 * C, :])
    c0_ref[...] = _S2 * (mb1_ref[...] + jnp.dot(pb2_ref[...], mw1c,
                                                preferred_element_type=_F32))


def _ka_body(tok_ref, call_ref, cblk_ref, n1g_ref, n1b_ref, wpc_ref, wq_ref,
             c0_ref, p_ref, t_ref, c_ref, idx_ref):
    b = pl.program_id(0)
    r = pl.program_id(1)

    call = call_ref[0]
    crb = cblk_ref[0]

    x = tok_ref[0]
    mu = jnp.mean(x, axis=1, keepdims=True)
    var = jnp.mean((x - mu) ** 2, axis=1, keepdims=True)
    tn = (x - mu) / jnp.sqrt(var + 1e-5) * n1g_ref[...] + n1b_ref[...]
    p_ref[0] = jnp.dot(tn, wpc_ref[...], preferred_element_type=_F32) \
        + c0_ref[...]
    q = jnp.dot(tn, wq_ref[...], preferred_element_type=_F32)
    hi = lax.bitcast_convert_type(
        q[:, 0:QW].astype(jnp.bfloat16).astype(_F32), jnp.int32)
    lo = lax.bitcast_convert_type(
        q[:, QW:C].astype(jnp.bfloat16).astype(_F32), jnp.int32)
    t_ref[0] = hi | lax.shift_right_logical(lo, 16)
    c_ref[0] = jnp.concatenate(
        [crb, jnp.zeros((RB, CW - CP), _F32)], axis=1)
    sqc = jnp.sum(crb * crb, axis=1, keepdims=True)
    ones_row = jnp.ones((1, CP), _F32)
    sqr = lax.dot_general(ones_row, call * call,
                          (((1,), (1,)), ((), ())),
                          preferred_element_type=_F32)
    cct = lax.dot_general(crb, call, (((1,), (1,)), ((), ())),
                          preferred_element_type=_F32)
    d2 = sqc + sqr - 2.0 * cct

    rows = lax.broadcasted_iota(jnp.int32, (RB, N), 0) + r * RB
    cols = lax.broadcasted_iota(jnp.int32, (RB, N), 1)
    d2 = jnp.where(rows == cols, 1e9, d2)
    bits = lax.bitcast_convert_type(d2, jnp.int32)
    keys = (bits & jnp.int32(-1024)) | cols

    fw = N // 4
    x0, x1 = keys[:, 0:fw], keys[:, fw:2 * fw]
    x2, x3 = keys[:, 2 * fw:3 * fw], keys[:, 3 * fw:4 * fw]
    a0, a1 = jnp.minimum(x0, x1), jnp.maximum(x0, x1)
    a2, a3 = jnp.minimum(x2, x3), jnp.maximum(x2, x3)
    b0, b2 = jnp.minimum(a0, a2), jnp.maximum(a0, a2)
    b1, b3 = jnp.minimum(a1, a3), jnp.maximum(a1, a3)
    c1, c2 = jnp.minimum(b1, b2), jnp.maximum(b1, b2)
    cur, n1, n2, n3 = b0, c1, c2, b3

    imax = jnp.int32(2147483647)
    lane_k = lax.broadcasted_iota(jnp.int32, (RB, K), 1)
    acc = jnp.zeros((RB, K), jnp.int32)
    for t in range(K):
        m = jnp.min(cur, axis=1, keepdims=True)
        acc = jnp.where(lane_k == t, (m & 1023) + b * N, acc)
        eq = cur == m
        cur = jnp.where(eq, n1, cur)
        n1 = jnp.where(eq, n2, n1)
        n2 = jnp.where(eq, n3, n2)
        n3 = jnp.where(eq, imax, n3)
    idx_ref[0] = acc


@functools.lru_cache(maxsize=None)
def _make_sc_gather(nktot):
    info = plsc.get_sparse_core_info()
    nc, ns = info.num_cores, info.num_subcores
    nw = nc * ns
    b_per_w = nktot // nw
    nch = b_per_w // GCH
    mesh = plsc.VectorSubcoreMesh(core_axis_name="c", subcore_axis_name="s")

    @functools.partial(
        pl.kernel, mesh=mesh,
        out_type=[jax.ShapeDtypeStruct((nktot, QW), jnp.int32),
                  jax.ShapeDtypeStruct((nktot, CW), _F32)],
        scratch_types=[
            pltpu.VMEM((GCH,), jnp.int32),
            pltpu.VMEM((GCH,), jnp.int32),
            pltpu.VMEM((GCH, QW), jnp.int32),
            pltpu.VMEM((GCH, QW), jnp.int32),
            pltpu.VMEM((GCH, CW), _F32),
            pltpu.VMEM((GCH, CW), _F32),
            pltpu.SemaphoreType.DMA,
            pltpu.SemaphoreType.DMA,
            pltpu.SemaphoreType.DMA,
            pltpu.SemaphoreType.DMA,
        ],
    )
    def gather_k(qtab, ctab, idx, qg, cg, idx0, idx1, qb0, qb1, cb0, cb1,
                 sg0, sg1, so0, so1):
        wid = lax.axis_index("s") * nc + lax.axis_index("c")
        base = wid * b_per_w
        idxb, qb, cb = [idx0, idx1], [qb0, qb1], [cb0, cb1]
        sg, so = [sg0, sg1], [so0, so1]
        gq = [None, None]
        gc = [None, None]
        oq = [None, None]
        oc = [None, None]

        def start_out(j, off):
            gq[j].wait()
            gc[j].wait()
            oq[j] = pltpu.async_copy(qb[j], qg.at[pl.ds(off, GCH)], so[j])
            oc[j] = pltpu.async_copy(cb[j], cg.at[pl.ds(off, GCH)], so[j])

        for ch in range(nch):
            bi = ch % 2
            if oq[bi] is not None:
                oq[bi].wait()
                oc[bi].wait()
            pltpu.sync_copy(idx.at[pl.ds(base + ch * GCH, GCH)], idxb[bi])
            gq[bi] = pltpu.async_copy(qtab.at[idxb[bi]], qb[bi], sg[bi])
            gc[bi] = pltpu.async_copy(ctab.at[idxb[bi]], cb[bi], sg[bi])
            if ch >= 1:
                start_out((ch - 1) % 2, base + (ch - 1) * GCH)
        last = (nch - 1) % 2
        start_out(last, base + (nch - 1) * GCH)
        oq[0].wait()
        oc[0].wait()
        oq[1].wait()
        oc[1].wait()

    return gather_k


def _sc_gather(qtab, ctab, idxf):
    return _make_sc_gather(idxf.shape[0])(qtab, ctab, idxf)


def _kc_body(tok_ref, p_ref, cpd_ref, qg_ref, cg_ref, pw1p_ref, w_ref,
             mw2_ref, fw1_ref, fw2_ref, mb2_ref, n2g_ref, n2b_ref,
             fb1_ref, fb2_ref, out_ref):
    cg3 = cg_ref[:, 0:CP].reshape(NB, K, CP)
    dxyz = (cg3 - cpd_ref[...][:, None, :]).reshape(NB * K, CP)
    posh = _g(jnp.dot(dxyz, pw1p_ref[...], preferred_element_type=_F32))
    u = jnp.dot(posh, w_ref[...], preferred_element_type=_F32)
    v = qg_ref[...]
    qhi = lax.bitcast_convert_type(v & jnp.int32(-65536), _F32)
    qlo = lax.bitcast_convert_type(lax.shift_left(v, 16), _F32)
    qg = jnp.concatenate([qhi, qlo], axis=1)
    h3 = _g(u.reshape(NB, K, C) + qg.reshape(NB, K, C)
            + p_ref[...][:, None, :])
    hs = jnp.sum(h3, axis=1)
    t = tok_ref[...] + jnp.dot(hs, mw2_ref[...],
                               preferred_element_type=_F32) + mb2_ref[...]

    mu = jnp.mean(t, axis=1, keepdims=True)
    var = jnp.mean((t - mu) ** 2, axis=1, keepdims=True)
    h = (t - mu) / jnp.sqrt(var + 1e-5) * n2g_ref[...] + n2b_ref[...]
    f = _g(jnp.dot(h, fw1_ref[...], preferred_element_type=_F32)
           + fb1_ref[...])
    out_ref[...] = t + jnp.dot(f, fw2_ref[...],
                               preferred_element_type=_F32) + fb2_ref[...]


def kernel(tokens, centers, n1_g, n1_b, pw1, pb1, pw2, pb2, mw1, mb1, mw2,
           mb2, n2_g, n2_b, fw1, fb1, fw2, fb2):
    row = lambda v: v.reshape(1, -1)
    cpad = jnp.pad(centers, ((0, 0), (0, 0), (0, CP - 3)))
    pw1p = _S2 * jnp.pad(pw1, ((0, CP - 3), (0, 0)))
    pw1p = pw1p.at[3].set(_S2 * pb1)
    cneg = cpad.at[:, :, 3].set(-1.0)
    wq = _S2 * mw1[C:2 * C]
    mw2_s = (_S2 / K) * mw2
    fw1_s = _S2 * fw1
    fb1_s = _S2 * fb1
    fw2_s = _S2 * fw2

    w_fused, wpc, c0 = pl.pallas_call(
        _prep_body,
        out_shape=[jax.ShapeDtypeStruct((C, C), _F32),
                   jax.ShapeDtypeStruct((C, C), _F32),
                   jax.ShapeDtypeStruct((1, C), _F32)],
    )(pw2, mw1, row(mb1), row(pb2))

    BH = B // 2
    nth = BH * N
    nkh = nth * K

    def run_half(tok_h, cpad_h, cneg_h):
        full = lambda s: pl.BlockSpec(s, lambda b, r: (0, 0))
        p_arr, t_arr, c_tab, idx = pl.pallas_call(
            _ka_body,
            grid=(BH, N // RB),
            in_specs=[
                pl.BlockSpec((1, RB, C), lambda b, r: (b, r, 0)),
                pl.BlockSpec((1, N, CP), lambda b, r: (b, 0, 0)),
                pl.BlockSpec((1, RB, CP), lambda b, r: (b, r, 0)),
                full((1, C)), full((1, C)),
                full((C, C)), full((C, C)), full((1, C)),
            ],
            out_specs=[
                pl.BlockSpec((1, RB, C), lambda b, r: (b, r, 0)),
                pl.BlockSpec((1, RB, QW), lambda b, r: (b, r, 0)),
                pl.BlockSpec((1, RB, CW), lambda b, r: (b, r, 0)),
                pl.BlockSpec((1, RB, K), lambda b, r: (b, r, 0)),
            ],
            out_shape=[jax.ShapeDtypeStruct((BH, N, C), _F32),
                       jax.ShapeDtypeStruct((BH, N, QW), jnp.int32),
                       jax.ShapeDtypeStruct((BH, N, CW), _F32),
                       jax.ShapeDtypeStruct((BH, N, K), jnp.int32)],
        )(tok_h, cpad_h, cpad_h, row(n1_g), row(n1_b), wpc, wq, c0)

        qg, cg = _sc_gather(t_arr.reshape(nth, QW),
                            c_tab.reshape(nth, CW), idx.reshape(nkh))

        wfull = lambda s: pl.BlockSpec(s, lambda i: (0, 0))
        out = pl.pallas_call(
            _kc_body,
            grid=(nth // NB,),
            in_specs=[
                pl.BlockSpec((NB, C), lambda i: (i, 0)),
                pl.BlockSpec((NB, C), lambda i: (i, 0)),
                pl.BlockSpec((NB, CP), lambda i: (i, 0)),
                pl.BlockSpec((NB * K, QW), lambda i: (i, 0)),
                pl.BlockSpec((NB * K, CW), lambda i: (i, 0)),
                wfull((CP, C)), wfull((C, C)), wfull((C, C)),
                wfull((C, 4 * C)), wfull((4 * C, C)),
                wfull((1, C)), wfull((1, C)), wfull((1, C)),
                wfull((1, 4 * C)), wfull((1, C)),
            ],
            out_specs=pl.BlockSpec((NB, C), lambda i: (i, 0)),
            out_shape=jax.ShapeDtypeStruct((nth, C), _F32),
        )(tok_h.reshape(nth, C), p_arr.reshape(nth, C),
          cneg_h.reshape(nth, CP), qg, cg, pw1p, w_fused, mw2_s, fw1_s,
          fw2_s, row(mb2), row(n2_g), row(n2_b), row(fb1_s), row(fb2))
        return out

    parts = [run_half(tokens[i:i + BH], cpad[i:i + BH], cneg[i:i + BH])
             for i in range(0, B, BH)]
    return jnp.concatenate(parts, axis=0).reshape(B, N, C)


# speedup vs baseline: 1.0875x; 1.0624x over previous
import functools

import jax
import jax.numpy as jnp
from jax import lax
from jax.experimental import pallas as pl
from jax.experimental.pallas import tpu as pltpu
from jax.experimental.pallas import tpu_sc as plsc

B, N, C, K = 4, 1024, 256, 16
CP = 16
QW = 128
CW = 128
RB = 256
NB = 256
NTOT = B * N
NKTOT = B * N * K
GCH = 128

_F32 = jnp.float32
_S2 = 0.7071067811865476


def _g(y):
    return y * (1.0 + lax.erf(y))


def _prep_body(pw2_ref, mw1_ref, mb1_ref, pb2_ref, w_ref, wpc_ref, c0_ref):
    mw1c = mw1_ref[2 * C:3 * C, :]
    w_ref[...] = 0.5 * jnp.dot(pw2_ref[...], mw1c,
                               preferred_element_type=_F32)
    wpc_ref[...] = _S2 * (mw1_ref[0:C, :] - mw1_ref[C:2 * C, :])
    c0_ref[...] = _S2 * (mb1_ref[...] + jnp.dot(pb2_ref[...], mw1c,
                                                preferred_element_type=_F32))


def _ka_body(tok_ref, call_ref, cblk_ref, n1g_ref, n1b_ref, wpc_ref, wq_ref,
             c0_ref, p_ref, t_ref, c_ref, idx_ref):
    b = pl.program_id(0)
    r = pl.program_id(1)

    call = call_ref[0]
    crb = cblk_ref[0]

    x = tok_ref[0]
    mu = jnp.mean(x, axis=1, keepdims=True)
    var = jnp.mean((x - mu) ** 2, axis=1, keepdims=True)
    tn = (x - mu) / jnp.sqrt(var + 1e-5) * n1g_ref[...] + n1b_ref[...]
    p_ref[0] = jnp.dot(tn, wpc_ref[...], preferred_element_type=_F32) \
        + c0_ref[...]
    q = jnp.dot(tn, wq_ref[...], preferred_element_type=_F32)
    hi = lax.bitcast_convert_type(
        q[:, 0:QW].astype(jnp.bfloat16).astype(_F32), jnp.int32)
    lo = lax.bitcast_convert_type(
        q[:, QW:C].astype(jnp.bfloat16).astype(_F32), jnp.int32)
    t_ref[0] = hi | lax.shift_right_logical(lo, 16)
    c_ref[0] = jnp.concatenate(
        [crb, jnp.zeros((RB, CW - CP), _F32)], axis=1)
    sqc = jnp.sum(crb * crb, axis=1, keepdims=True)
    ones_row = jnp.ones((1, CP), _F32)
    sqr = lax.dot_general(ones_row, call * call,
                          (((1,), (1,)), ((), ())),
                          preferred_element_type=_F32)
    cct = lax.dot_general(crb, call, (((1,), (1,)), ((), ())),
                          preferred_element_type=_F32)
    d2 = sqc + sqr - 2.0 * cct

    rows = lax.broadcasted_iota(jnp.int32, (RB, N), 0) + r * RB
    cols = lax.broadcasted_iota(jnp.int32, (RB, N), 1)
    d2 = jnp.where(rows == cols, 1e9, d2)
    bits = lax.bitcast_convert_type(d2, jnp.int32)
    keys = (bits & jnp.int32(-1024)) | cols

    fw = N // 4
    x0, x1 = keys[:, 0:fw], keys[:, fw:2 * fw]
    x2, x3 = keys[:, 2 * fw:3 * fw], keys[:, 3 * fw:4 * fw]
    a0, a1 = jnp.minimum(x0, x1), jnp.maximum(x0, x1)
    a2, a3 = jnp.minimum(x2, x3), jnp.maximum(x2, x3)
    b0, b2 = jnp.minimum(a0, a2), jnp.maximum(a0, a2)
    b1, b3 = jnp.minimum(a1, a3), jnp.maximum(a1, a3)
    c1, c2 = jnp.minimum(b1, b2), jnp.maximum(b1, b2)
    cur, n1, n2, n3 = b0, c1, c2, b3

    imax = jnp.int32(2147483647)
    lane_k = lax.broadcasted_iota(jnp.int32, (RB, K), 1)
    acc = jnp.zeros((RB, K), jnp.int32)
    for t in range(K):
        m = jnp.min(cur, axis=1, keepdims=True)
        acc = jnp.where(lane_k == t, (m & 1023) + b * N, acc)
        eq = cur == m
        cur = jnp.where(eq, n1, cur)
        n1 = jnp.where(eq, n2, n1)
        n2 = jnp.where(eq, n3, n2)
        n3 = jnp.where(eq, imax, n3)
    idx_ref[0] = acc


@functools.lru_cache(maxsize=None)
def _make_sc_gather(nktot):
    info = plsc.get_sparse_core_info()
    nc, ns = info.num_cores, info.num_subcores
    nw = nc * ns
    b_per_w = nktot // nw
    nch = b_per_w // GCH
    mesh = plsc.VectorSubcoreMesh(core_axis_name="c", subcore_axis_name="s")

    @functools.partial(
        pl.kernel, mesh=mesh,
        out_type=[jax.ShapeDtypeStruct((nktot, QW), jnp.int32),
                  jax.ShapeDtypeStruct((nktot, CW), _F32)],
        scratch_types=[
            pltpu.VMEM((GCH,), jnp.int32),
            pltpu.VMEM((GCH,), jnp.int32),
            pltpu.VMEM((GCH, QW), jnp.int32),
            pltpu.VMEM((GCH, QW), jnp.int32),
            pltpu.VMEM((GCH, CW), _F32),
            pltpu.VMEM((GCH, CW), _F32),
            pltpu.SemaphoreType.DMA,
            pltpu.SemaphoreType.DMA,
            pltpu.SemaphoreType.DMA,
            pltpu.SemaphoreType.DMA,
        ],
    )
    def gather_k(qtab, ctab, idx, qg, cg, idx0, idx1, qb0, qb1, cb0, cb1,
                 sg0, sg1, so0, so1):
        wid = lax.axis_index("s") * nc + lax.axis_index("c")
        base = wid * b_per_w
        idxb, qb, cb = [idx0, idx1], [qb0, qb1], [cb0, cb1]
        sg, so = [sg0, sg1], [so0, so1]
        gq = [None, None]
        gc = [None, None]
        oq = [None, None]
        oc = [None, None]

        def start_out(j, off):
            gq[j].wait()
            gc[j].wait()
            oq[j] = pltpu.async_copy(qb[j], qg.at[pl.ds(off, GCH)], so[j])
            oc[j] = pltpu.async_copy(cb[j], cg.at[pl.ds(off, GCH)], so[j])

        for ch in range(nch):
            bi = ch % 2
            if oq[bi] is not None:
                oq[bi].wait()
                oc[bi].wait()
            pltpu.sync_copy(idx.at[pl.ds(base + ch * GCH, GCH)], idxb[bi])
            gq[bi] = pltpu.async_copy(qtab.at[idxb[bi]], qb[bi], sg[bi])
            gc[bi] = pltpu.async_copy(ctab.at[idxb[bi]], cb[bi], sg[bi])
            if ch >= 1:
                start_out((ch - 1) % 2, base + (ch - 1) * GCH)
        last = (nch - 1) % 2
        start_out(last, base + (nch - 1) * GCH)
        oq[0].wait()
        oc[0].wait()
        oq[1].wait()
        oc[1].wait()

    return gather_k


def _sc_gather(qtab, ctab, idxf):
    return _make_sc_gather(idxf.shape[0])(qtab, ctab, idxf)


def _kc_body(tok_ref, p_ref, cpd_ref, qg_ref, cg_ref, pw1p_ref, w_ref,
             mw2_ref, fw1_ref, fw2_ref, mb2_ref, n2g_ref, n2b_ref,
             fb1_ref, fb2_ref, out_ref):
    e3 = jnp.where(lax.broadcasted_iota(jnp.int32, (NB, CP), 1) == 3,
                   1.0, 0.0).astype(_F32)
    cg3 = cg_ref[:, 0:CP].reshape(NB, K, CP)
    dxyz = (cg3 - (cpd_ref[...] - e3)[:, None, :]).reshape(NB * K, CP)
    posh = _g(jnp.dot(dxyz, pw1p_ref[...], preferred_element_type=_F32))
    u = jnp.dot(posh, w_ref[...], preferred_element_type=_F32)
    v = qg_ref[...]
    qhi = lax.bitcast_convert_type(v & jnp.int32(-65536), _F32)
    qlo = lax.bitcast_convert_type(lax.shift_left(v, 16), _F32)
    qg = jnp.concatenate([qhi, qlo], axis=1)
    h3 = _g(u.reshape(NB, K, C) + qg.reshape(NB, K, C)
            + p_ref[...][:, None, :])
    hs = jnp.sum(h3, axis=1)
    t = tok_ref[...] + jnp.dot(hs, mw2_ref[...],
                               preferred_element_type=_F32) + mb2_ref[...]

    mu = jnp.mean(t, axis=1, keepdims=True)
    var = jnp.mean((t - mu) ** 2, axis=1, keepdims=True)
    h = (t - mu) / jnp.sqrt(var + 1e-5) * n2g_ref[...] + n2b_ref[...]
    f = _g(jnp.dot(h, fw1_ref[...], preferred_element_type=_F32)
           + fb1_ref[...])
    out_ref[...] = t + jnp.dot(f, fw2_ref[...],
                               preferred_element_type=_F32) + fb2_ref[...]


def kernel(tokens, centers, n1_g, n1_b, pw1, pb1, pw2, pb2, mw1, mb1, mw2,
           mb2, n2_g, n2_b, fw1, fb1, fw2, fb2):
    row = lambda v: v.reshape(1, -1)
    cpad = jnp.pad(centers, ((0, 0), (0, 0), (0, CP - 3)))
    pw1p = _S2 * jnp.pad(pw1, ((0, CP - 3), (0, 0)))
    pw1p = pw1p.at[3].set(_S2 * pb1)
    wq = _S2 * mw1[C:2 * C]
    mw2_s = (_S2 / K) * mw2
    fw1_s = _S2 * fw1
    fb1_s = _S2 * fb1
    fw2_s = _S2 * fw2

    w_fused, wpc, c0 = pl.pallas_call(
        _prep_body,
        out_shape=[jax.ShapeDtypeStruct((C, C), _F32),
                   jax.ShapeDtypeStruct((C, C), _F32),
                   jax.ShapeDtypeStruct((1, C), _F32)],
    )(pw2, mw1, row(mb1), row(pb2))

    BH = B // 2
    nth = BH * N
    nkh = nth * K

    def run_half(tok_h, cpad_h):
        full = lambda s: pl.BlockSpec(s, lambda b, r: (0, 0))
        p_arr, t_arr, c_tab, idx = pl.pallas_call(
            _ka_body,
            grid=(BH, N // RB),
            in_specs=[
                pl.BlockSpec((1, RB, C), lambda b, r: (b, r, 0)),
                pl.BlockSpec((1, N, CP), lambda b, r: (b, 0, 0)),
                pl.BlockSpec((1, RB, CP), lambda b, r: (b, r, 0)),
                full((1, C)), full((1, C)),
                full((C, C)), full((C, C)), full((1, C)),
            ],
            out_specs=[
                pl.BlockSpec((1, RB, C), lambda b, r: (b, r, 0)),
                pl.BlockSpec((1, RB, QW), lambda b, r: (b, r, 0)),
                pl.BlockSpec((1, RB, CW), lambda b, r: (b, r, 0)),
                pl.BlockSpec((1, RB, K), lambda b, r: (b, r, 0)),
            ],
            out_shape=[jax.ShapeDtypeStruct((BH, N, C), _F32),
                       jax.ShapeDtypeStruct((BH, N, QW), jnp.int32),
                       jax.ShapeDtypeStruct((BH, N, CW), _F32),
                       jax.ShapeDtypeStruct((BH, N, K), jnp.int32)],
        )(tok_h, cpad_h, cpad_h, row(n1_g), row(n1_b), wpc, wq, c0)

        qg, cg = _sc_gather(t_arr.reshape(nth, QW),
                            c_tab.reshape(nth, CW), idx.reshape(nkh))

        wfull = lambda s: pl.BlockSpec(s, lambda i: (0, 0))
        out = pl.pallas_call(
            _kc_body,
            grid=(nth // NB,),
            in_specs=[
                pl.BlockSpec((NB, C), lambda i: (i, 0)),
                pl.BlockSpec((NB, C), lambda i: (i, 0)),
                pl.BlockSpec((NB, CP), lambda i: (i, 0)),
                pl.BlockSpec((NB * K, QW), lambda i: (i, 0)),
                pl.BlockSpec((NB * K, CW), lambda i: (i, 0)),
                wfull((CP, C)), wfull((C, C)), wfull((C, C)),
                wfull((C, 4 * C)), wfull((4 * C, C)),
                wfull((1, C)), wfull((1, C)), wfull((1, C)),
                wfull((1, 4 * C)), wfull((1, C)),
            ],
            out_specs=pl.BlockSpec((NB, C), lambda i: (i, 0)),
            out_shape=jax.ShapeDtypeStruct((nth, C), _F32),
        )(tok_h.reshape(nth, C), p_arr.reshape(nth, C),
          cpad_h.reshape(nth, CP), qg, cg, pw1p, w_fused, mw2_s, fw1_s,
          fw2_s, row(mb2), row(n2_g), row(n2_b), row(fb1_s), row(fb2))
        return out

    parts = [run_half(tokens[i:i + BH], cpad[i:i + BH])
             for i in range(0, B, BH)]
    return jnp.concatenate(parts, axis=0).reshape(B, N, C)
